# baseline (speedup 1.0000x reference)
import functools

import jax
import jax.numpy as jnp
from jax import lax
from jax.experimental import pallas as pl
from jax.experimental.pallas import tpu as pltpu
from jax.experimental.pallas import tpu_sc as plsc

EMBEDDING_DIM = 32

_info = plsc.get_sparse_core_info()
_NC, _NS = _info.num_cores, _info.num_subcores
_NW = _NC * _NS
_L = 16

_CHUNK = 1024


def _make_gather(B: int, NB: int, D: int):
    assert B % (_NW * _CHUNK) == 0 and NB % _CHUNK == 0
    n_chunks = (B // _NW) // _CHUNK
    chunks_per_row = NB // _CHUNK
    t_slab = D * NB
    n_dt = D // 8
    stage_sz = _CHUNK * D // n_dt
    mesh = plsc.VectorSubcoreMesh(core_axis_name="c", subcore_axis_name="s")

    @functools.partial(
        pl.kernel,
        out_type=jax.ShapeDtypeStruct((B * D,), jnp.float32),
        mesh=mesh,
        scratch_types=[
            pltpu.VMEM((_CHUNK,), jnp.int32),
            pltpu.VMEM((_CHUNK,), jnp.int32),
            pltpu.VMEM((_CHUNK, D), jnp.float32),
            pltpu.VMEM((_CHUNK, D), jnp.float32),
            pltpu.VMEM((_CHUNK * D,), jnp.float32),
            pltpu.SemaphoreType.DMA,
            pltpu.SemaphoreType.DMA,
            pltpu.SemaphoreType.DMA,
        ],
        compiler_params=pltpu.CompilerParams(
            use_tc_tiling_on_sc=False, needs_layout_passes=False
        ),
    )
    def gather_kernel(table_hbm, idx_hbm, out_hbm, idx0, idx1, rows0, rows1,
                      stage, gs0, gs1, ws):
        wid = lax.axis_index("s") * _NC + lax.axis_index("c")
        g_base = wid * n_chunks
        idx_b = (idx0, idx1)
        rows_b = (rows0, rows1)
        gs = (gs0, gs1)
        lanes = jnp.arange(_L, dtype=jnp.int32)

        def fire(g, b):
            pltpu.sync_copy(idx_hbm.at[pl.ds(g * _CHUNK, _CHUNK)], idx_b[b])
            pltpu.async_copy(table_hbm.at[idx_b[b]], rows_b[b], gs[b])

        def drain(g, b, not_first):
            pltpu.make_async_copy(table_hbm.at[idx_b[b]], rows_b[b], gs[b]).wait()

            @pl.when(not_first)
            def _():
                for p in range(_CHUNK * D // 1024):
                    pltpu.make_async_copy(
                        stage.at[pl.ds(p * 1024, 1024)],
                        out_hbm.at[pl.ds(p * 1024, 1024)],
                        ws,
                    ).wait()

            rows = rows_b[b]

            l128 = lanes << 7

            def jb_body(jb, carry):
                j0 = jb * _L
                jaddr = ((j0 >> 7) << 12) | (j0 & 127)
                for d0 in (0, _L):
                    a = l128 + (jaddr + d0 * 128)
                    for jj in range(_L):
                        v = rows[j0 + jj, pl.ds(d0, _L)]
                        plsc.store_scatter(stage, [a], v)
                        a = a + 1
                return carry

            lax.fori_loop(0, _CHUNK // _L, jb_body, 0, unroll=False)

            t = g // chunks_per_row
            b0 = (g % chunks_per_row) * _CHUNK
            for bb in range(_CHUNK // 128):
                for k in range(n_dt):
                    off = t * t_slab + k * (8 * NB) + b0 * 8 + bb * 1024
                    pltpu.async_copy(
                        stage.at[pl.ds(bb * (D * 128) + k * 1024, 1024)],
                        out_hbm.at[pl.ds(off, 1024)],
                        ws,
                    )

        fire(g_base, 0)

        def body(p, carry):
            g0 = g_base + 2 * p
            fire(g0 + 1, 1)
            drain(g0, 0, p > 0)

            @pl.when(2 * p + 2 < n_chunks)
            def _():
                fire(g0 + 2, 0)

            drain(g0 + 1, 1, p >= 0)
            return carry

        lax.fori_loop(0, n_chunks // 2, body, 0, unroll=False)
        for p in range(_CHUNK * D // 1024):
            pltpu.make_async_copy(
                stage.at[pl.ds(p * 1024, 1024)],
                out_hbm.at[pl.ds(p * 1024, 1024)],
                ws,
            ).wait()

    return gather_kernel


def kernel(x, table):
    NB, NT = x.shape
    B = NB * NT
    D = EMBEDDING_DIM
    idx = x.T.reshape(-1).astype(jnp.int32)
    flat = _make_gather(B, NB, D)(table, idx)
    out5 = flat.reshape(NT, D // 8, NB // 128, 8, 128)
    out = out5.transpose(2, 4, 0, 1, 3).reshape(NB, NT, D)
    return out


# speedup vs baseline: 1.8920x; 1.8920x over previous
import functools

import jax
import jax.numpy as jnp
from jax import lax
from jax.experimental import pallas as pl
from jax.experimental.pallas import tpu as pltpu
from jax.experimental.pallas import tpu_sc as plsc

EMBEDDING_DIM = 32

_info = plsc.get_sparse_core_info()
_NC, _NS = _info.num_cores, _info.num_subcores
_NW = _NC * _NS
_L = 16

_CHUNK = 1024


def _make_gather(B: int, NB: int, D: int):
    assert B % (_NW * _CHUNK) == 0 and NB % _CHUNK == 0
    n_chunks = (B // _NW) // _CHUNK
    chunks_per_row = NB // _CHUNK
    t_slab = D * NB
    n_dt = D // 8
    stage_sz = _CHUNK * D // n_dt
    mesh = plsc.VectorSubcoreMesh(core_axis_name="c", subcore_axis_name="s")

    @functools.partial(
        pl.kernel,
        out_type=jax.ShapeDtypeStruct((B * D,), jnp.float32),
        mesh=mesh,
        scratch_types=[
            pltpu.VMEM((_CHUNK,), jnp.int32),
            pltpu.VMEM((_CHUNK,), jnp.int32),
            pltpu.VMEM((_CHUNK, D), jnp.float32),
            pltpu.VMEM((_CHUNK, D), jnp.float32),
            pltpu.VMEM((_CHUNK * D,), jnp.float32),
            pltpu.SemaphoreType.DMA,
            pltpu.SemaphoreType.DMA,
            pltpu.SemaphoreType.DMA,
        ],
        compiler_params=pltpu.CompilerParams(
            use_tc_tiling_on_sc=False, needs_layout_passes=False
        ),
    )
    def gather_kernel(table_hbm, idx_hbm, out_hbm, idx0, idx1, rows0, rows1,
                      stage, gs0, gs1, ws):
        wid = lax.axis_index("s") * _NC + lax.axis_index("c")
        g_base = wid * n_chunks
        idx_b = (idx0, idx1)
        rows_b = (rows0, rows1)
        gs = (gs0, gs1)
        lanes = jnp.arange(_L, dtype=jnp.int32)

        def fire(g, b):
            pltpu.sync_copy(idx_hbm.at[pl.ds(g * _CHUNK, _CHUNK)], idx_b[b])
            pltpu.async_copy(table_hbm.at[idx_b[b]], rows_b[b], gs[b])

        def drain(g, b, not_first):
            pltpu.make_async_copy(table_hbm.at[idx_b[b]], rows_b[b], gs[b]).wait()

            @pl.when(not_first)
            def _():
                for p in range(_CHUNK * D // 1024):
                    pltpu.make_async_copy(
                        stage.at[pl.ds(p * 1024, 1024)],
                        out_hbm.at[pl.ds(p * 1024, 1024)],
                        ws,
                    ).wait()

            rows = rows_b[b]

            def jb_body(jb, carry):
                j0v = lanes + jb * _L
                jaddr = ((j0v >> 7) << 12) | (j0v & 127)
                r = lanes
                for _k in range(_L):
                    a0 = jaddr + (r << 7)
                    c16 = r + _L
                    v0 = plsc.load_gather(rows, [j0v, r])
                    plsc.store_scatter(stage, [a0], v0)
                    v1 = plsc.load_gather(rows, [j0v, c16])
                    plsc.store_scatter(stage, [a0 + _L * 128], v1)
                    r = (r + 1) & (_L - 1)
                return carry

            lax.fori_loop(0, _CHUNK // _L, jb_body, 0, unroll=4)

            t = g // chunks_per_row
            b0 = (g % chunks_per_row) * _CHUNK
            for bb in range(_CHUNK // 128):
                for k in range(n_dt):
                    off = t * t_slab + k * (8 * NB) + b0 * 8 + bb * 1024
                    pltpu.async_copy(
                        stage.at[pl.ds(bb * (D * 128) + k * 1024, 1024)],
                        out_hbm.at[pl.ds(off, 1024)],
                        ws,
                    )

        fire(g_base, 0)

        def body(p, carry):
            g0 = g_base + 2 * p
            fire(g0 + 1, 1)
            drain(g0, 0, p > 0)

            @pl.when(2 * p + 2 < n_chunks)
            def _():
                fire(g0 + 2, 0)

            drain(g0 + 1, 1, p >= 0)
            return carry

        lax.fori_loop(0, n_chunks // 2, body, 0, unroll=False)
        for p in range(_CHUNK * D // 1024):
            pltpu.make_async_copy(
                stage.at[pl.ds(p * 1024, 1024)],
                out_hbm.at[pl.ds(p * 1024, 1024)],
                ws,
            ).wait()

    return gather_kernel


def kernel(x, table):
    NB, NT = x.shape
    B = NB * NT
    D = EMBEDDING_DIM
    idx = x.T.reshape(-1).astype(jnp.int32)
    flat = _make_gather(B, NB, D)(table, idx)
    out5 = flat.reshape(NT, D // 8, NB // 128, 8, 128)
    out = out5.transpose(2, 4, 0, 1, 3).reshape(NB, NT, D)
    return out
